# baseline (speedup 1.0000x reference)
import functools

import jax
import jax.numpy as jnp
from jax import lax
from jax.experimental import pallas as pl
from jax.experimental.pallas import tpu as pltpu
from jax.experimental.pallas import tpu_sc as plsc

_N = 10000
_E = 160000
_D = 256
_DW = _D // 2
_SCALE = (128.0 ** 0.5) * 4.0
_INV_SCALE = 1.0 / _SCALE

_NC = 2
_NS = 16
_NW = _NC * _NS
_C = 64
_EP = 163840
_GRP = _C // 16
_Q0 = 5120
_Q1 = 5120
_QMAX = max(_Q0, _Q1)
_O1 = 16 * _Q0
_NF1 = (_E - _O1) // _Q1
_Q1T = _E - _O1 - _NF1 * _Q1
_O1T = _O1 + _NF1 * _Q1

_TCB = 2000


def _tc_body(xs_ref, xd_ref, wq_ref, bq_ref, wk_ref, bk_ref,
             yt_ref, fsb_ref, tstd_ref):
    wq = wq_ref[...]
    wk = wk_ref[...]
    bq = bq_ref[...]
    bk = bk_ref[...]
    a = lax.dot_general(wq, wk, (((0,), (0,)), ((), ())),
                        preferred_element_type=jnp.float32)
    xd = xd_ref[...]
    yt = lax.dot_general(xd, a, (((1,), (1,)), ((), ())),
                         preferred_element_type=jnp.float32)
    yt_ref[...] = (yt * _INV_SCALE).astype(jnp.bfloat16)
    ws = lax.dot_general(wq, bk, (((0,), (0,)), ((), ())))
    wd = lax.dot_general(wk, bq, (((0,), (0,)), ((), ())))
    c = jnp.sum(bq * bk)
    xs = xs_ref[...]
    fsb_ref[...] = xs.astype(jnp.bfloat16)
    ts = (lax.dot_general(xs, ws[:, None], (((1,), (0,)), ((), ())),
                          preferred_element_type=jnp.float32) + c) * _INV_SCALE
    td = lax.dot_general(xd, wd[:, None], (((1,), (0,)), ((), ())),
                         preferred_element_type=jnp.float32) * _INV_SCALE
    col = lax.broadcasted_iota(jnp.int32, (_TCB, 8), 1)
    tstd_ref[...] = jnp.where(col == 0, ts, jnp.where(col == 1, td, 0.0))


def _tc_tables(feat_src, feat_dst, wq, bq, wk, bk):
    grid = _N // _TCB
    return pl.pallas_call(
        _tc_body,
        grid=(grid,),
        in_specs=[
            pl.BlockSpec((_TCB, _D), lambda i: (i, 0)),
            pl.BlockSpec((_TCB, _D), lambda i: (i, 0)),
            pl.BlockSpec((512, _D), lambda i: (0, 0)),
            pl.BlockSpec((512,), lambda i: (0,)),
            pl.BlockSpec((512, _D), lambda i: (0, 0)),
            pl.BlockSpec((512,), lambda i: (0,)),
        ],
        out_specs=[
            pl.BlockSpec((_TCB, _D), lambda i: (i, 0)),
            pl.BlockSpec((_TCB, _D), lambda i: (i, 0)),
            pl.BlockSpec((_TCB, 8), lambda i: (i, 0)),
        ],
        out_shape=[
            jax.ShapeDtypeStruct((_N, _D), jnp.bfloat16),
            jax.ShapeDtypeStruct((_N, _D), jnp.bfloat16),
            jax.ShapeDtypeStruct((_N, 8), jnp.float32),
        ],
    )(feat_src, feat_dst, wq, bq, wk, bk)


def _sc_body(fs_hbm, yt_hbm, ts_hbm, td_hbm, src_hbm, out_hbm,
             idxs_v, idxd_v, rx0_v, ry0_v, rx1_v, ry1_v, ts_v, td_v, sc_v,
             semx0, semy0, semx1, semy1):
    cid = lax.axis_index("c")
    sid = lax.axis_index("s")

    @pl.when(cid == 0)
    def _():
        bo = pl.multiple_of(sid * _Q0, 8)
        pltpu.sync_copy(src_hbm.at[0, pl.ds(bo, _Q0)], idxs_v.at[pl.ds(0, _Q0)])
        pltpu.sync_copy(src_hbm.at[1, pl.ds(bo, _Q0)], idxd_v.at[pl.ds(0, _Q0)])

    @pl.when((cid == 1) & (sid < _NF1))
    def _():
        bo = pl.multiple_of(_O1 + sid * _Q1, 8)
        pltpu.sync_copy(src_hbm.at[0, pl.ds(bo, _Q1)], idxs_v.at[pl.ds(0, _Q1)])
        pltpu.sync_copy(src_hbm.at[1, pl.ds(bo, _Q1)], idxd_v.at[pl.ds(0, _Q1)])

    @pl.when((cid == 1) & (sid == _NF1))
    def _():
        bo = pl.multiple_of(_O1T, 8)
        pltpu.sync_copy(src_hbm.at[0, pl.ds(bo, _Q1T)], idxs_v.at[pl.ds(0, _Q1T)])
        pltpu.sync_copy(src_hbm.at[1, pl.ds(bo, _Q1T)], idxd_v.at[pl.ds(0, _Q1T)])

    pltpu.sync_copy(ts_hbm, ts_v)
    pltpu.sync_copy(td_hbm, td_v)

    nchunks = jnp.where(
        cid == 0, _Q0 // _C,
        jnp.where(sid < _NF1, _Q1 // _C,
                  jnp.where(sid == _NF1, _Q1T // _C, 0)))

    iota16 = lax.iota(jnp.int32, 16)
    zero16 = jnp.zeros((16,), jnp.float32)

    def start(j, rx, ry, semx, semy):
        pltpu.async_copy(fs_hbm.at[idxs_v.at[pl.ds(j * _C, _C)]], rx, semx)
        pltpu.async_copy(yt_hbm.at[idxd_v.at[pl.ds(j * _C, _C)]], ry, semy)

    def wait(j, rx, ry, semx, semy):
        pltpu.make_async_copy(fs_hbm.at[idxs_v.at[pl.ds(j * _C, _C)]],
                              rx, semx).wait()
        pltpu.make_async_copy(yt_hbm.at[idxd_v.at[pl.ds(j * _C, _C)]],
                              ry, semy).wait()

    def compute(j, rx_v, ry_v):
        for g in range(_GRP):
            rows16 = iota16 + (g * 16)
            src16 = idxs_v[pl.ds(j * _C + g * 16, 16)]
            dst16 = idxd_v[pl.ds(j * _C + g * 16, 16)]
            acc0 = plsc.load_gather(ts_v, [src16]) + plsc.load_gather(td_v, [dst16])

            @plsc.parallel_loop(0, _DW, step=2, unroll=4,
                                carry=(acc0, zero16, zero16, zero16))
            def kbody(k, accs):
                base = jnp.full((16,), k, jnp.int32) + iota16
                out = list(accs)
                for u in range(2):
                    kvu = (base + u) & (_DW - 1)
                    gxi = plsc.load_gather(rx_v, [rows16, kvu])
                    gyi = plsc.load_gather(ry_v, [rows16, kvu])
                    xa, xb = plsc.unpack(plsc.bitcast(gxi, jnp.bfloat16),
                                         format=plsc.PackFormat.INTERLEAVED)
                    ya, yb = plsc.unpack(plsc.bitcast(gyi, jnp.bfloat16),
                                         format=plsc.PackFormat.INTERLEAVED)
                    out[2 * u] = out[2 * u] + xa * ya
                    out[2 * u + 1] = out[2 * u + 1] + xb * yb
                return tuple(out)

            a0, a1, a2, a3 = kbody
            sc_v[pl.ds(j * _C + g * 16, 16)] = (a0 + a1) + (a2 + a3)

    @pl.when(nchunks > 0)
    def _():
        start(0, rx0_v, ry0_v, semx0, semy0)

        def pair_body(i, carry):
            j0 = 2 * i
            j1 = j0 + 1
            start(j1, rx1_v, ry1_v, semx1, semy1)
            wait(j0, rx0_v, ry0_v, semx0, semy0)
            compute(j0, rx0_v, ry0_v)

            @pl.when(j1 + 1 < nchunks)
            def _():
                start(j1 + 1, rx0_v, ry0_v, semx0, semy0)

            wait(j1, rx1_v, ry1_v, semx1, semy1)
            compute(j1, rx1_v, ry1_v)
            return carry

        lax.fori_loop(0, nchunks // 2, pair_body, 0)

    @pl.when(cid == 0)
    def _():
        bo = pl.multiple_of(sid * _Q0, 8)
        pltpu.sync_copy(sc_v.at[pl.ds(0, _Q0)], out_hbm.at[pl.ds(bo, _Q0)])

    @pl.when((cid == 1) & (sid < _NF1))
    def _():
        bo = pl.multiple_of(_O1 + sid * _Q1, 8)
        pltpu.sync_copy(sc_v.at[pl.ds(0, _Q1)], out_hbm.at[pl.ds(bo, _Q1)])

    @pl.when((cid == 1) & (sid == _NF1))
    def _():
        bo = pl.multiple_of(_O1T, 8)
        pltpu.sync_copy(sc_v.at[pl.ds(0, _Q1T)], out_hbm.at[pl.ds(bo, _Q1T)])


_sc_edge_scores = functools.partial(
    pl.kernel,
    out_type=jax.ShapeDtypeStruct((_E,), jnp.float32),
    mesh=plsc.VectorSubcoreMesh(core_axis_name="c", subcore_axis_name="s",
                                num_cores=_NC, num_subcores=_NS),
    compiler_params=pltpu.CompilerParams(needs_layout_passes=False),
    scratch_types=[
        pltpu.VMEM((_QMAX,), jnp.int32),
        pltpu.VMEM((_QMAX,), jnp.int32),
        pltpu.VMEM((_C, _DW), jnp.int32),
        pltpu.VMEM((_C, _DW), jnp.int32),
        pltpu.VMEM((_C, _DW), jnp.int32),
        pltpu.VMEM((_C, _DW), jnp.int32),
        pltpu.VMEM((_N,), jnp.float32),
        pltpu.VMEM((_N,), jnp.float32),
        pltpu.VMEM((_QMAX,), jnp.float32),
        pltpu.SemaphoreType.DMA,
        pltpu.SemaphoreType.DMA,
        pltpu.SemaphoreType.DMA,
        pltpu.SemaphoreType.DMA,
    ],
)(_sc_body)


@jax.jit
def kernel(feat_src, feat_dst, edge_index, Wq, bq, Wk, bk):
    ei = edge_index.astype(jnp.int32)
    yt, fsb, tstd = _tc_tables(feat_src, feat_dst, Wq, bq, Wk, bk)
    fsp = lax.bitcast_convert_type(fsb.reshape(_N, _DW, 2), jnp.int32)
    ytp = lax.bitcast_convert_type(yt.reshape(_N, _DW, 2), jnp.int32)
    scores = _sc_edge_scores(fsp, ytp, tstd[:, 0], tstd[:, 1], ei)
    return scores.reshape(_E, 1)


# speedup vs baseline: 1.7385x; 1.7385x over previous
import functools

import jax
import jax.numpy as jnp
from jax import lax
from jax.experimental import pallas as pl
from jax.experimental.pallas import tpu as pltpu
from jax.experimental.pallas import tpu_sc as plsc

_N = 10000
_E = 160000
_D = 256
_SCALE = (128.0 ** 0.5) * 4.0
_INV_SCALE = 1.0 / _SCALE

_NC = 2
_NS = 16
_NW = _NC * _NS
_C = 64
_EP = 163840
_GRP = _C // 16
_Q0 = 5120
_Q1 = 5120
_QMAX = max(_Q0, _Q1)
_O1 = 16 * _Q0
_NF1 = (_E - _O1) // _Q1
_Q1T = _E - _O1 - _NF1 * _Q1
_O1T = _O1 + _NF1 * _Q1

_TCB = 1000


def _tc_body(xs_ref, xd_ref, wq_ref, bq_ref, wk_ref, bk_ref, yt_ref, tstd_ref):
    wq = wq_ref[...]
    wk = wk_ref[...]
    bq = bq_ref[...]
    bk = bk_ref[...]
    a = lax.dot_general(wq, wk, (((0,), (0,)), ((), ())),
                        preferred_element_type=jnp.float32)
    xd = xd_ref[...]
    yt = lax.dot_general(xd, a, (((1,), (1,)), ((), ())),
                         preferred_element_type=jnp.float32)
    yt_ref[...] = yt * _INV_SCALE
    ws = lax.dot_general(wq, bk, (((0,), (0,)), ((), ())))
    wd = lax.dot_general(wk, bq, (((0,), (0,)), ((), ())))
    c = jnp.sum(bq * bk)
    xs = xs_ref[...]
    ts = (lax.dot_general(xs, ws[:, None], (((1,), (0,)), ((), ())),
                          preferred_element_type=jnp.float32) + c) * _INV_SCALE
    td = lax.dot_general(xd, wd[:, None], (((1,), (0,)), ((), ())),
                         preferred_element_type=jnp.float32) * _INV_SCALE
    col = lax.broadcasted_iota(jnp.int32, (_TCB, 8), 1)
    tstd_ref[...] = jnp.where(col == 0, ts, jnp.where(col == 1, td, 0.0))


def _tc_tables(feat_src, feat_dst, wq, bq, wk, bk):
    grid = _N // _TCB
    return pl.pallas_call(
        _tc_body,
        grid=(grid,),
        in_specs=[
            pl.BlockSpec((_TCB, _D), lambda i: (i, 0)),
            pl.BlockSpec((_TCB, _D), lambda i: (i, 0)),
            pl.BlockSpec((512, _D), lambda i: (0, 0)),
            pl.BlockSpec((512,), lambda i: (0,)),
            pl.BlockSpec((512, _D), lambda i: (0, 0)),
            pl.BlockSpec((512,), lambda i: (0,)),
        ],
        out_specs=[
            pl.BlockSpec((_TCB, _D), lambda i: (i, 0)),
            pl.BlockSpec((_TCB, 8), lambda i: (i, 0)),
        ],
        out_shape=[
            jax.ShapeDtypeStruct((_N, _D), jnp.float32),
            jax.ShapeDtypeStruct((_N, 8), jnp.float32),
        ],
    )(feat_src, feat_dst, wq, bq, wk, bk)


def _sc_body(fs_hbm, yt_hbm, ts_hbm, td_hbm, src_hbm, out_hbm,
             idxs_v, idxd_v, rx0_v, ry0_v, rx1_v, ry1_v, ts_v, td_v, sc_v,
             semx0, semy0, semx1, semy1):
    cid = lax.axis_index("c")
    sid = lax.axis_index("s")

    @pl.when(cid == 0)
    def _():
        bo = pl.multiple_of(sid * _Q0, 8)
        pltpu.sync_copy(src_hbm.at[0, pl.ds(bo, _Q0)], idxs_v.at[pl.ds(0, _Q0)])
        pltpu.sync_copy(src_hbm.at[1, pl.ds(bo, _Q0)], idxd_v.at[pl.ds(0, _Q0)])

    @pl.when((cid == 1) & (sid < _NF1))
    def _():
        bo = pl.multiple_of(_O1 + sid * _Q1, 8)
        pltpu.sync_copy(src_hbm.at[0, pl.ds(bo, _Q1)], idxs_v.at[pl.ds(0, _Q1)])
        pltpu.sync_copy(src_hbm.at[1, pl.ds(bo, _Q1)], idxd_v.at[pl.ds(0, _Q1)])

    @pl.when((cid == 1) & (sid == _NF1))
    def _():
        bo = pl.multiple_of(_O1T, 8)
        pltpu.sync_copy(src_hbm.at[0, pl.ds(bo, _Q1T)], idxs_v.at[pl.ds(0, _Q1T)])
        pltpu.sync_copy(src_hbm.at[1, pl.ds(bo, _Q1T)], idxd_v.at[pl.ds(0, _Q1T)])

    pltpu.sync_copy(ts_hbm, ts_v)
    pltpu.sync_copy(td_hbm, td_v)

    nchunks = jnp.where(
        cid == 0, _Q0 // _C,
        jnp.where(sid < _NF1, _Q1 // _C,
                  jnp.where(sid == _NF1, _Q1T // _C, 0)))

    iota16 = lax.iota(jnp.int32, 16)
    zero16 = jnp.zeros((16,), jnp.float32)

    def start(j, rx, ry, semx, semy):
        pltpu.async_copy(fs_hbm.at[idxs_v.at[pl.ds(j * _C, _C)]], rx, semx)
        pltpu.async_copy(yt_hbm.at[idxd_v.at[pl.ds(j * _C, _C)]], ry, semy)

    def wait(j, rx, ry, semx, semy):
        pltpu.make_async_copy(fs_hbm.at[idxs_v.at[pl.ds(j * _C, _C)]],
                              rx, semx).wait()
        pltpu.make_async_copy(yt_hbm.at[idxd_v.at[pl.ds(j * _C, _C)]],
                              ry, semy).wait()

    def compute(j, rx_v, ry_v):
        for g in range(_GRP):
            rows16 = iota16 + (g * 16)
            src16 = idxs_v[pl.ds(j * _C + g * 16, 16)]
            dst16 = idxd_v[pl.ds(j * _C + g * 16, 16)]
            acc0 = plsc.load_gather(ts_v, [src16]) + plsc.load_gather(td_v, [dst16])

            @plsc.parallel_loop(0, _D, step=4, unroll=4,
                                carry=(acc0, zero16, zero16, zero16))
            def kbody(k, accs):
                base = jnp.full((16,), k, jnp.int32) + iota16
                out = []
                for u in range(4):
                    kvu = (base + u) & (_D - 1)
                    gx = plsc.load_gather(rx_v, [rows16, kvu])
                    gy = plsc.load_gather(ry_v, [rows16, kvu])
                    out.append(accs[u] + gx * gy)
                return tuple(out)

            a0, a1, a2, a3 = kbody
            sc_v[pl.ds(j * _C + g * 16, 16)] = (a0 + a1) + (a2 + a3)

    @pl.when(nchunks > 0)
    def _():
        start(0, rx0_v, ry0_v, semx0, semy0)

        def pair_body(i, carry):
            j0 = 2 * i
            j1 = j0 + 1
            start(j1, rx1_v, ry1_v, semx1, semy1)
            wait(j0, rx0_v, ry0_v, semx0, semy0)
            compute(j0, rx0_v, ry0_v)

            @pl.when(j1 + 1 < nchunks)
            def _():
                start(j1 + 1, rx0_v, ry0_v, semx0, semy0)

            wait(j1, rx1_v, ry1_v, semx1, semy1)
            compute(j1, rx1_v, ry1_v)
            return carry

        lax.fori_loop(0, nchunks // 2, pair_body, 0)

    @pl.when(cid == 0)
    def _():
        bo = pl.multiple_of(sid * _Q0, 8)
        pltpu.sync_copy(sc_v.at[pl.ds(0, _Q0)], out_hbm.at[pl.ds(bo, _Q0)])

    @pl.when((cid == 1) & (sid < _NF1))
    def _():
        bo = pl.multiple_of(_O1 + sid * _Q1, 8)
        pltpu.sync_copy(sc_v.at[pl.ds(0, _Q1)], out_hbm.at[pl.ds(bo, _Q1)])

    @pl.when((cid == 1) & (sid == _NF1))
    def _():
        bo = pl.multiple_of(_O1T, 8)
        pltpu.sync_copy(sc_v.at[pl.ds(0, _Q1T)], out_hbm.at[pl.ds(bo, _Q1T)])


_sc_edge_scores = functools.partial(
    pl.kernel,
    out_type=jax.ShapeDtypeStruct((_E,), jnp.float32),
    mesh=plsc.VectorSubcoreMesh(core_axis_name="c", subcore_axis_name="s",
                                num_cores=_NC, num_subcores=_NS),
    compiler_params=pltpu.CompilerParams(needs_layout_passes=False),
    scratch_types=[
        pltpu.VMEM((_QMAX,), jnp.int32),
        pltpu.VMEM((_QMAX,), jnp.int32),
        pltpu.VMEM((_C, _D), jnp.float32),
        pltpu.VMEM((_C, _D), jnp.float32),
        pltpu.VMEM((_C, _D), jnp.float32),
        pltpu.VMEM((_C, _D), jnp.float32),
        pltpu.VMEM((_N,), jnp.float32),
        pltpu.VMEM((_N,), jnp.float32),
        pltpu.VMEM((_QMAX,), jnp.float32),
        pltpu.SemaphoreType.DMA,
        pltpu.SemaphoreType.DMA,
        pltpu.SemaphoreType.DMA,
        pltpu.SemaphoreType.DMA,
    ],
)(_sc_body)


@jax.jit
def kernel(feat_src, feat_dst, edge_index, Wq, bq, Wk, bk):
    ei = edge_index.astype(jnp.int32)
    yt, tstd = _tc_tables(feat_src, feat_dst, Wq, bq, Wk, bk)
    scores = _sc_edge_scores(feat_src, yt, tstd[:, 0], tstd[:, 1], ei)
    return scores.reshape(_E, 1)


# speedup vs baseline: 1.7664x; 1.0160x over previous
import functools

import jax
import jax.numpy as jnp
from jax import lax
from jax.experimental import pallas as pl
from jax.experimental.pallas import tpu as pltpu
from jax.experimental.pallas import tpu_sc as plsc

_N = 10000
_E = 160000
_D = 256
_SCALE = (128.0 ** 0.5) * 4.0
_INV_SCALE = 1.0 / _SCALE

_NC = 2
_NS = 16
_NW = _NC * _NS
_C = 80
_EP = 163840
_GRP = _C // 16
_Q0 = 5120
_Q1 = 5120
_QMAX = max(_Q0, _Q1)
_O1 = 16 * _Q0
_NF1 = (_E - _O1) // _Q1
_Q1T = _E - _O1 - _NF1 * _Q1
_O1T = _O1 + _NF1 * _Q1

_TCB = 1000


def _tc_body(xs_ref, xd_ref, wq_ref, bq_ref, wk_ref, bk_ref, yt_ref, tstd_ref):
    wq = wq_ref[...]
    wk = wk_ref[...]
    bq = bq_ref[...]
    bk = bk_ref[...]
    a = lax.dot_general(wq, wk, (((0,), (0,)), ((), ())),
                        preferred_element_type=jnp.float32)
    xd = xd_ref[...]
    yt = lax.dot_general(xd, a, (((1,), (1,)), ((), ())),
                         preferred_element_type=jnp.float32)
    yt_ref[...] = yt * _INV_SCALE
    ws = lax.dot_general(wq, bk, (((0,), (0,)), ((), ())))
    wd = lax.dot_general(wk, bq, (((0,), (0,)), ((), ())))
    c = jnp.sum(bq * bk)
    xs = xs_ref[...]
    ts = (lax.dot_general(xs, ws[:, None], (((1,), (0,)), ((), ())),
                          preferred_element_type=jnp.float32) + c) * _INV_SCALE
    td = lax.dot_general(xd, wd[:, None], (((1,), (0,)), ((), ())),
                         preferred_element_type=jnp.float32) * _INV_SCALE
    col = lax.broadcasted_iota(jnp.int32, (_TCB, 8), 1)
    tstd_ref[...] = jnp.where(col == 0, ts, jnp.where(col == 1, td, 0.0))


def _tc_tables(feat_src, feat_dst, wq, bq, wk, bk):
    grid = _N // _TCB
    return pl.pallas_call(
        _tc_body,
        grid=(grid,),
        in_specs=[
            pl.BlockSpec((_TCB, _D), lambda i: (i, 0)),
            pl.BlockSpec((_TCB, _D), lambda i: (i, 0)),
            pl.BlockSpec((512, _D), lambda i: (0, 0)),
            pl.BlockSpec((512,), lambda i: (0,)),
            pl.BlockSpec((512, _D), lambda i: (0, 0)),
            pl.BlockSpec((512,), lambda i: (0,)),
        ],
        out_specs=[
            pl.BlockSpec((_TCB, _D), lambda i: (i, 0)),
            pl.BlockSpec((_TCB, 8), lambda i: (i, 0)),
        ],
        out_shape=[
            jax.ShapeDtypeStruct((_N, _D), jnp.float32),
            jax.ShapeDtypeStruct((_N, 8), jnp.float32),
        ],
    )(feat_src, feat_dst, wq, bq, wk, bk)


def _sc_body(fs_hbm, yt_hbm, ts_hbm, td_hbm, src_hbm, out_hbm,
             idxs_v, idxd_v, rx0_v, ry0_v, rx1_v, ry1_v, ts_v, td_v, sc_v,
             semx0, semy0, semx1, semy1):
    cid = lax.axis_index("c")
    sid = lax.axis_index("s")

    @pl.when(cid == 0)
    def _():
        bo = pl.multiple_of(sid * _Q0, 8)
        pltpu.sync_copy(src_hbm.at[0, pl.ds(bo, _Q0)], idxs_v.at[pl.ds(0, _Q0)])
        pltpu.sync_copy(src_hbm.at[1, pl.ds(bo, _Q0)], idxd_v.at[pl.ds(0, _Q0)])

    @pl.when((cid == 1) & (sid < _NF1))
    def _():
        bo = pl.multiple_of(_O1 + sid * _Q1, 8)
        pltpu.sync_copy(src_hbm.at[0, pl.ds(bo, _Q1)], idxs_v.at[pl.ds(0, _Q1)])
        pltpu.sync_copy(src_hbm.at[1, pl.ds(bo, _Q1)], idxd_v.at[pl.ds(0, _Q1)])

    @pl.when((cid == 1) & (sid == _NF1))
    def _():
        bo = pl.multiple_of(_O1T, 8)
        pltpu.sync_copy(src_hbm.at[0, pl.ds(bo, _Q1T)], idxs_v.at[pl.ds(0, _Q1T)])
        pltpu.sync_copy(src_hbm.at[1, pl.ds(bo, _Q1T)], idxd_v.at[pl.ds(0, _Q1T)])

    pltpu.sync_copy(ts_hbm, ts_v)
    pltpu.sync_copy(td_hbm, td_v)

    nchunks = jnp.where(
        cid == 0, _Q0 // _C,
        jnp.where(sid < _NF1, _Q1 // _C,
                  jnp.where(sid == _NF1, _Q1T // _C, 0)))

    iota16 = lax.iota(jnp.int32, 16)
    zero16 = jnp.zeros((16,), jnp.float32)

    def start(j, rx, ry, semx, semy):
        pltpu.async_copy(fs_hbm.at[idxs_v.at[pl.ds(j * _C, _C)]], rx, semx)
        pltpu.async_copy(yt_hbm.at[idxd_v.at[pl.ds(j * _C, _C)]], ry, semy)

    def wait(j, rx, ry, semx, semy):
        pltpu.make_async_copy(fs_hbm.at[idxs_v.at[pl.ds(j * _C, _C)]],
                              rx, semx).wait()
        pltpu.make_async_copy(yt_hbm.at[idxd_v.at[pl.ds(j * _C, _C)]],
                              ry, semy).wait()

    def compute(j, rx_v, ry_v):
        for g in range(_GRP):
            rows16 = iota16 + (g * 16)
            src16 = idxs_v[pl.ds(j * _C + g * 16, 16)]
            dst16 = idxd_v[pl.ds(j * _C + g * 16, 16)]
            acc0 = plsc.load_gather(ts_v, [src16]) + plsc.load_gather(td_v, [dst16])

            @plsc.parallel_loop(0, _D, step=4, unroll=4,
                                carry=(acc0, zero16, zero16, zero16))
            def kbody(k, accs):
                base = jnp.full((16,), k, jnp.int32) + iota16
                out = []
                for u in range(4):
                    kvu = (base + u) & (_D - 1)
                    gx = plsc.load_gather(rx_v, [rows16, kvu])
                    gy = plsc.load_gather(ry_v, [rows16, kvu])
                    out.append(accs[u] + gx * gy)
                return tuple(out)

            a0, a1, a2, a3 = kbody
            sc_v[pl.ds(j * _C + g * 16, 16)] = (a0 + a1) + (a2 + a3)

    @pl.when(nchunks > 0)
    def _():
        start(0, rx0_v, ry0_v, semx0, semy0)

        def pair_body(i, carry):
            j0 = 2 * i
            j1 = j0 + 1
            start(j1, rx1_v, ry1_v, semx1, semy1)
            wait(j0, rx0_v, ry0_v, semx0, semy0)
            compute(j0, rx0_v, ry0_v)

            @pl.when(j1 + 1 < nchunks)
            def _():
                start(j1 + 1, rx0_v, ry0_v, semx0, semy0)

            wait(j1, rx1_v, ry1_v, semx1, semy1)
            compute(j1, rx1_v, ry1_v)
            return carry

        lax.fori_loop(0, nchunks // 2, pair_body, 0)

    @pl.when(cid == 0)
    def _():
        bo = pl.multiple_of(sid * _Q0, 8)
        pltpu.sync_copy(sc_v.at[pl.ds(0, _Q0)], out_hbm.at[pl.ds(bo, _Q0)])

    @pl.when((cid == 1) & (sid < _NF1))
    def _():
        bo = pl.multiple_of(_O1 + sid * _Q1, 8)
        pltpu.sync_copy(sc_v.at[pl.ds(0, _Q1)], out_hbm.at[pl.ds(bo, _Q1)])

    @pl.when((cid == 1) & (sid == _NF1))
    def _():
        bo = pl.multiple_of(_O1T, 8)
        pltpu.sync_copy(sc_v.at[pl.ds(0, _Q1T)], out_hbm.at[pl.ds(bo, _Q1T)])


_sc_edge_scores = functools.partial(
    pl.kernel,
    out_type=jax.ShapeDtypeStruct((_E,), jnp.float32),
    mesh=plsc.VectorSubcoreMesh(core_axis_name="c", subcore_axis_name="s",
                                num_cores=_NC, num_subcores=_NS),
    compiler_params=pltpu.CompilerParams(needs_layout_passes=False),
    scratch_types=[
        pltpu.VMEM((_QMAX,), jnp.int32),
        pltpu.VMEM((_QMAX,), jnp.int32),
        pltpu.VMEM((_C, _D), jnp.float32),
        pltpu.VMEM((_C, _D), jnp.float32),
        pltpu.VMEM((_C, _D), jnp.float32),
        pltpu.VMEM((_C, _D), jnp.float32),
        pltpu.VMEM((_N,), jnp.float32),
        pltpu.VMEM((_N,), jnp.float32),
        pltpu.VMEM((_QMAX,), jnp.float32),
        pltpu.SemaphoreType.DMA,
        pltpu.SemaphoreType.DMA,
        pltpu.SemaphoreType.DMA,
        pltpu.SemaphoreType.DMA,
    ],
)(_sc_body)


@jax.jit
def kernel(feat_src, feat_dst, edge_index, Wq, bq, Wk, bk):
    ei = edge_index.astype(jnp.int32)
    yt, tstd = _tc_tables(feat_src, feat_dst, Wq, bq, Wk, bk)
    scores = _sc_edge_scores(feat_src, yt, tstd[:, 0], tstd[:, 1], ei)
    return scores.reshape(_E, 1)


# speedup vs baseline: 1.7668x; 1.0002x over previous
import functools

import jax
import jax.numpy as jnp
from jax import lax
from jax.experimental import pallas as pl
from jax.experimental.pallas import tpu as pltpu
from jax.experimental.pallas import tpu_sc as plsc

_N = 10000
_E = 160000
_D = 256
_SCALE = (128.0 ** 0.5) * 4.0
_INV_SCALE = 1.0 / _SCALE

_NC = 2
_NS = 16
_NW = _NC * _NS
_C = 80
_GRP = _C // 16
_Q0 = 5120
_Q1 = 5120
_QMAX = max(_Q0, _Q1)
_O1 = 16 * _Q0
_NF1 = (_E - _O1) // _Q1
_Q1T = _E - _O1 - _NF1 * _Q1
_O1T = _O1 + _NF1 * _Q1

_TCB = 1000


def _tc_body(xs_ref, xd_ref, wq_ref, bq_ref, wk_ref, bk_ref, yt_ref, tstd_ref):
    wq = wq_ref[...]
    wk = wk_ref[...]
    bq = bq_ref[...]
    bk = bk_ref[...]
    a = lax.dot_general(wq, wk, (((0,), (0,)), ((), ())),
                        preferred_element_type=jnp.float32)
    xd = xd_ref[...]
    yt = lax.dot_general(xd, a, (((1,), (1,)), ((), ())),
                         preferred_element_type=jnp.float32)
    yt_ref[...] = yt * _INV_SCALE
    ws = lax.dot_general(wq, bk, (((0,), (0,)), ((), ())))
    wd = lax.dot_general(wk, bq, (((0,), (0,)), ((), ())))
    c = jnp.sum(bq * bk)
    xs = xs_ref[...]
    ts = (lax.dot_general(xs, ws[:, None], (((1,), (0,)), ((), ())),
                          preferred_element_type=jnp.float32) + c) * _INV_SCALE
    td = lax.dot_general(xd, wd[:, None], (((1,), (0,)), ((), ())),
                         preferred_element_type=jnp.float32) * _INV_SCALE
    col = lax.broadcasted_iota(jnp.int32, (_TCB, 8), 1)
    tstd_ref[...] = jnp.where(col == 0, ts, jnp.where(col == 1, td, 0.0))


def _tc_tables(feat_src, feat_dst, wq, bq, wk, bk):
    grid = _N // _TCB
    return pl.pallas_call(
        _tc_body,
        grid=(grid,),
        in_specs=[
            pl.BlockSpec((_TCB, _D), lambda i: (i, 0)),
            pl.BlockSpec((_TCB, _D), lambda i: (i, 0)),
            pl.BlockSpec((512, _D), lambda i: (0, 0)),
            pl.BlockSpec((512,), lambda i: (0,)),
            pl.BlockSpec((512, _D), lambda i: (0, 0)),
            pl.BlockSpec((512,), lambda i: (0,)),
        ],
        out_specs=[
            pl.BlockSpec((_TCB, _D), lambda i: (i, 0)),
            pl.BlockSpec((_TCB, 8), lambda i: (i, 0)),
        ],
        out_shape=[
            jax.ShapeDtypeStruct((_N, _D), jnp.float32),
            jax.ShapeDtypeStruct((_N, 8), jnp.float32),
        ],
    )(feat_src, feat_dst, wq, bq, wk, bk)


def _sc_body(fs_hbm, yt_hbm, ts_hbm, td_hbm, src_hbm, out_hbm,
             idxs_v, idxd_v, rx0_v, ry0_v, rx1_v, ry1_v, ts_v, td_v, sc_v,
             semx0, semy0, semx1, semy1):
    cid = lax.axis_index("c")
    sid = lax.axis_index("s")

    @pl.when(cid == 0)
    def _():
        bo = pl.multiple_of(sid * _Q0, 8)
        pltpu.sync_copy(src_hbm.at[0, pl.ds(bo, _Q0)], idxs_v.at[pl.ds(0, _Q0)])
        pltpu.sync_copy(src_hbm.at[1, pl.ds(bo, _Q0)], idxd_v.at[pl.ds(0, _Q0)])

    @pl.when((cid == 1) & (sid < _NF1))
    def _():
        bo = pl.multiple_of(_O1 + sid * _Q1, 8)
        pltpu.sync_copy(src_hbm.at[0, pl.ds(bo, _Q1)], idxs_v.at[pl.ds(0, _Q1)])
        pltpu.sync_copy(src_hbm.at[1, pl.ds(bo, _Q1)], idxd_v.at[pl.ds(0, _Q1)])

    @pl.when((cid == 1) & (sid == _NF1))
    def _():
        bo = pl.multiple_of(_O1T, 8)
        pltpu.sync_copy(src_hbm.at[0, pl.ds(bo, _Q1T)], idxs_v.at[pl.ds(0, _Q1T)])
        pltpu.sync_copy(src_hbm.at[1, pl.ds(bo, _Q1T)], idxd_v.at[pl.ds(0, _Q1T)])

    pltpu.sync_copy(ts_hbm, ts_v)
    pltpu.sync_copy(td_hbm, td_v)

    nchunks = jnp.where(
        cid == 0, _Q0 // _C,
        jnp.where(sid < _NF1, _Q1 // _C,
                  jnp.where(sid == _NF1, _Q1T // _C, 0)))

    iota16 = lax.iota(jnp.int32, 16)
    zero16 = jnp.zeros((16,), jnp.float32)

    def start(j, rx, ry, semx, semy):
        pltpu.async_copy(fs_hbm.at[idxs_v.at[pl.ds(j * _C, _C)]], rx, semx)
        pltpu.async_copy(yt_hbm.at[idxd_v.at[pl.ds(j * _C, _C)]], ry, semy)

    def wait(j, rx, ry, semx, semy):
        pltpu.make_async_copy(fs_hbm.at[idxs_v.at[pl.ds(j * _C, _C)]],
                              rx, semx).wait()
        pltpu.make_async_copy(yt_hbm.at[idxd_v.at[pl.ds(j * _C, _C)]],
                              ry, semy).wait()

    def compute(j, rx_v, ry_v):
        for g in range(_GRP):
            rows16 = iota16 + (g * 16)
            src16 = idxs_v[pl.ds(j * _C + g * 16, 16)]
            dst16 = idxd_v[pl.ds(j * _C + g * 16, 16)]
            acc0 = plsc.load_gather(ts_v, [src16]) + plsc.load_gather(td_v, [dst16])

            @plsc.parallel_loop(0, _D, step=4, unroll=4,
                                carry=(acc0, zero16, zero16, zero16))
            def kbody(k, accs):
                base = jnp.full((16,), k, jnp.int32) + iota16
                out = []
                for u in range(4):
                    kvu = (base + u) & (_D - 1)
                    gx = plsc.load_gather(rx_v, [rows16, kvu])
                    gy = plsc.load_gather(ry_v, [rows16, kvu])
                    out.append(accs[u] + gx * gy)
                return tuple(out)

            a0, a1, a2, a3 = kbody
            sc_v[pl.ds(j * _C + g * 16, 16)] = (a0 + a1) + (a2 + a3)

    @pl.when(nchunks > 0)
    def _():
        start(0, rx0_v, ry0_v, semx0, semy0)

        def pair_body(i, carry):
            j0 = 2 * i
            j1 = j0 + 1
            start(j1, rx1_v, ry1_v, semx1, semy1)
            wait(j0, rx0_v, ry0_v, semx0, semy0)
            compute(j0, rx0_v, ry0_v)

            @pl.when(j1 + 1 < nchunks)
            def _():
                start(j1 + 1, rx0_v, ry0_v, semx0, semy0)

            wait(j1, rx1_v, ry1_v, semx1, semy1)
            compute(j1, rx1_v, ry1_v)
            return carry

        lax.fori_loop(0, nchunks // 2, pair_body, 0)

    @pl.when(cid == 0)
    def _():
        bo = pl.multiple_of(sid * _Q0, 8)
        pltpu.sync_copy(sc_v.at[pl.ds(0, _Q0)], out_hbm.at[pl.ds(bo, _Q0)])

    @pl.when((cid == 1) & (sid < _NF1))
    def _():
        bo = pl.multiple_of(_O1 + sid * _Q1, 8)
        pltpu.sync_copy(sc_v.at[pl.ds(0, _Q1)], out_hbm.at[pl.ds(bo, _Q1)])

    @pl.when((cid == 1) & (sid == _NF1))
    def _():
        bo = pl.multiple_of(_O1T, 8)
        pltpu.sync_copy(sc_v.at[pl.ds(0, _Q1T)], out_hbm.at[pl.ds(bo, _Q1T)])


_sc_edge_scores = functools.partial(
    pl.kernel,
    out_type=jax.ShapeDtypeStruct((_E,), jnp.float32),
    mesh=plsc.VectorSubcoreMesh(core_axis_name="c", subcore_axis_name="s",
                                num_cores=_NC, num_subcores=_NS),
    compiler_params=pltpu.CompilerParams(needs_layout_passes=False),
    scratch_types=[
        pltpu.VMEM((_QMAX,), jnp.int32),
        pltpu.VMEM((_QMAX,), jnp.int32),
        pltpu.VMEM((_C, _D), jnp.float32),
        pltpu.VMEM((_C, _D), jnp.float32),
        pltpu.VMEM((_C, _D), jnp.float32),
        pltpu.VMEM((_C, _D), jnp.float32),
        pltpu.VMEM((_N,), jnp.float32),
        pltpu.VMEM((_N,), jnp.float32),
        pltpu.VMEM((_QMAX,), jnp.float32),
        pltpu.SemaphoreType.DMA,
        pltpu.SemaphoreType.DMA,
        pltpu.SemaphoreType.DMA,
        pltpu.SemaphoreType.DMA,
    ],
)(_sc_body)


@jax.jit
def kernel(feat_src, feat_dst, edge_index, Wq, bq, Wk, bk):
    ei = edge_index.astype(jnp.int32)
    yt, tstd = _tc_tables(feat_src, feat_dst, Wq, bq, Wk, bk)
    scores = _sc_edge_scores(feat_src, yt, tstd[:, 0], tstd[:, 1], ei)
    return scores.reshape(_E, 1)
